# baseline (speedup 1.0000x reference)
import functools

import jax
from jax import lax
import jax.numpy as jnp
from jax.experimental import pallas as pl
from jax.experimental.pallas import tpu as pltpu
from jax.experimental.pallas import tpu_sc as plsc

_E = 8
_EPS = 0.2
_LANES = 16
_WPAD = 128
_NW = 32
_C = 3584

_SORT_NET = [
    (0, 1), (2, 3), (4, 5), (6, 7),
    (0, 2), (1, 3), (4, 6), (5, 7),
    (1, 2), (5, 6), (0, 4), (3, 7),
    (1, 5), (2, 6),
    (1, 4), (3, 6),
    (2, 4), (3, 5),
    (3, 4),
]


def _routing_body(x_ref, wsel_ref, bsel_ref, weff_ref, cost_ref):
    x = x_ref[...]
    logits = jax.lax.dot_general(
        x, wsel_ref[...], (((1,), (1,)), ((), ())),
        preferred_element_type=jnp.float32) + bsel_ref[...]
    m = jnp.max(logits, axis=1, keepdims=True)
    ex = jnp.exp(logits - m)
    w = ex / jnp.sum(ex, axis=1, keepdims=True)

    ws = [w[:, j:j + 1] for j in range(_E)]
    ids = [jnp.full(ws[0].shape, j, dtype=jnp.int32) for j in range(_E)]
    for a, b in _SORT_NET:
        wa, wb = ws[a], ws[b]
        ia, ib = ids[a], ids[b]
        swap = (wb > wa) | ((wb == wa) & (ib < ia))
        ws[a] = jnp.where(swap, wb, wa)
        ws[b] = jnp.where(swap, wa, wb)
        ids[a] = jnp.where(swap, ib, ia)
        ids[b] = jnp.where(swap, ia, ib)

    cum = [ws[0]]
    for k in range(1, _E):
        cum.append(cum[-1] + ws[k])

    lim = jnp.float32(1.0 - _EPS)
    sp = []
    for k in range(_E):
        nxt = cum[k + 1] if k < _E - 1 else jnp.full(cum[0].shape, 1.0, jnp.float32)
        sp.append(jax.nn.relu(jnp.minimum(nxt, lim) - cum[k]))

    cost = sp[_E - 1]
    for k in range(_E - 1):
        cost = cost + jnp.where(sp[k + 1] > 0, jnp.float32(1.0), sp[k])

    weff_cols = []
    for k in range(_E):
        col = jnp.zeros(cum[0].shape, jnp.float32)
        for j in range(_E):
            col = jnp.where(ids[k] == j, sp[j], col)
        weff_cols.append(col)
    zero = jnp.zeros(cum[0].shape, jnp.float32)
    weff_cols.extend([zero] * (_WPAD - _E))

    weff_ref[...] = jnp.concatenate(weff_cols, axis=1)
    cost_ref[...] = cost


def _combine_body(weff_ref, x_ref, wexp_ref, bexp_ref, out_ref):
    e = pl.program_id(1)
    d = jax.lax.dot_general(
        x_ref[...], wexp_ref[0], (((1,), (1,)), ((), ())),
        preferred_element_type=jnp.float32)
    weff = weff_ref[...]
    lane = jax.lax.broadcasted_iota(jnp.int32, weff.shape, 1)
    wcol = jnp.sum(jnp.where(lane == e, weff, 0.0), axis=1, keepdims=True)
    contrib = wcol * (d + bexp_ref[0])

    @pl.when(e == 0)
    def _init():
        out_ref[...] = contrib

    @pl.when(e != 0)
    def _acc():
        out_ref[...] += contrib


def _dense_combine(weff, xb, wexp_b, bexp, n, nin, nout):
    ctb = 2048
    return pl.pallas_call(
        _combine_body,
        grid=(n // ctb, _E),
        in_specs=[
            pl.BlockSpec((ctb, _WPAD), lambda t, e: (t, 0)),
            pl.BlockSpec((ctb, nin), lambda t, e: (t, 0)),
            pl.BlockSpec((1, nout, nin), lambda t, e: (e, 0, 0)),
            pl.BlockSpec((1, 1, nout), lambda t, e: (e, 0, 0)),
        ],
        out_specs=pl.BlockSpec((ctb, nout), lambda t, e: (t, 0)),
        out_shape=jax.ShapeDtypeStruct((n, nout), jnp.float32),
        compiler_params=pltpu.CompilerParams(
            dimension_semantics=("parallel", "arbitrary")),
    )(weff, xb, wexp_b, bexp.reshape(_E, 1, nout))


def _wid():
    return lax.axis_index("s") * 2 + lax.axis_index("c")


def _sc_mesh():
    return plsc.VectorSubcoreMesh(core_axis_name="c", subcore_axis_name="s")


def _make_compact(n):
    nv = n // _LANES

    @functools.partial(
        pl.kernel,
        mesh=_sc_mesh(),
        out_type=[
            jax.ShapeDtypeStruct((_C,), jnp.int32),
            jax.ShapeDtypeStruct((n,), jnp.int32),
            jax.ShapeDtypeStruct((_LANES,), jnp.int32),
        ],
        scratch_types=[
            pltpu.VMEM((n,), jnp.float32),
            pltpu.VMEM((_C,), jnp.int32),
            pltpu.VMEM((n,), jnp.int32),
            pltpu.VMEM((_LANES,), jnp.int32),
        ],
        compiler_params=pltpu.CompilerParams(needs_layout_passes=False),
    )
    def compact(cost_hbm, idx_hbm, g_hbm, cnt_hbm, cost_v, idx_v, g_v, cnt_v):
        @pl.when(_wid() == 0)
        def _():
            pltpu.sync_copy(cost_hbm, cost_v)

            def init_body(j, carry):
                idx_v[pl.ds(j * _LANES, _LANES)] = jnp.zeros(
                    (_LANES,), jnp.int32)
                return carry
            lax.fori_loop(0, _C // _LANES, init_body, jnp.int32(0))

            def scan_body(i, offv):
                cvec = cost_v[pl.ds(i * _LANES, _LANES)]
                m = cvec > 0.0
                base = lax.broadcast(i * _LANES, (_LANES,))
                ids = lax.iota(jnp.int32, _LANES) + base
                cum = plsc.cumsum(m.astype(jnp.int32))
                pos = offv + cum - jnp.ones((_LANES,), jnp.int32)
                cap = jnp.full((_LANES,), _C, jnp.int32)
                m_ok = m & (pos < cap)
                plsc.store_scatter(idx_v, [pos], ids, mask=m_ok)
                zrow = cap + (ids & jnp.full((_LANES,), 255, jnp.int32))
                g_v[pl.ds(i * _LANES, _LANES)] = jnp.where(m_ok, pos, zrow)
                return offv + plsc.all_reduce_population_count(m)
            total = lax.fori_loop(0, nv, scan_body,
                                  jnp.zeros((_LANES,), jnp.int32))

            cnt_v[...] = total
            pltpu.sync_copy(idx_v, idx_hbm)
            pltpu.sync_copy(g_v, g_hbm)
            pltpu.sync_copy(cnt_v, cnt_hbm)

    return compact


def _make_gather(n, nin):
    pw = _C // _NW

    @functools.partial(
        pl.kernel,
        mesh=_sc_mesh(),
        out_type=[
            jax.ShapeDtypeStruct((_C, nin), jnp.float32),
            jax.ShapeDtypeStruct((_C, _WPAD), jnp.float32),
        ],
        scratch_types=[
            pltpu.VMEM((pw,), jnp.int32),
            pltpu.VMEM((pw, nin), jnp.float32),
            pltpu.VMEM((pw, _WPAD), jnp.float32),
            pltpu.SemaphoreType.DMA,
            pltpu.SemaphoreType.DMA,
        ],
    )
    def gather(x_hbm, weff_hbm, idx_hbm, xc_hbm, wc_hbm,
               idx_v, rows_v, wrow_v, sem1, sem2):
        base = _wid() * pw
        pltpu.sync_copy(idx_hbm.at[pl.ds(base, pw)], idx_v)
        cp1 = pltpu.async_copy(x_hbm.at[idx_v], rows_v, sem1)
        cp2 = pltpu.async_copy(weff_hbm.at[idx_v], wrow_v, sem2)
        cp1.wait()
        cp2.wait()
        pltpu.sync_copy(rows_v, xc_hbm.at[pl.ds(base, pw)])
        pltpu.sync_copy(wrow_v, wc_hbm.at[pl.ds(base, pw)])

    return gather


def _make_finalize(n, nout):
    pw = n // _NW
    chunk = 32
    nchunk = pw // chunk

    @functools.partial(
        pl.kernel,
        mesh=_sc_mesh(),
        out_type=jax.ShapeDtypeStruct((n, nout), jnp.float32),
        scratch_types=[
            pltpu.VMEM((pw,), jnp.int32),
            pltpu.VMEM((chunk, nout), jnp.float32),
            pltpu.VMEM((chunk, nout), jnp.float32),
            pltpu.SemaphoreType.DMA,
            pltpu.SemaphoreType.DMA,
        ],
    )
    def finalize(yc_hbm, g_hbm, out_hbm, g_v, buf0, buf1, sem0, sem1):
        base = _wid() * pw
        pltpu.sync_copy(g_hbm.at[pl.ds(base, pw)], g_v)
        bufs = (buf0, buf1)
        sems = (sem0, sem1)
        cps = [None, None]
        cps[0] = pltpu.async_copy(
            yc_hbm.at[g_v.at[pl.ds(0, chunk)]], buf0, sem0)
        for c in range(nchunk):
            nxt = c + 1
            if nxt < nchunk:
                cps[nxt % 2] = pltpu.async_copy(
                    yc_hbm.at[g_v.at[pl.ds(nxt * chunk, chunk)]],
                    bufs[nxt % 2], sems[nxt % 2])
            cps[c % 2].wait()
            pltpu.sync_copy(
                bufs[c % 2], out_hbm.at[pl.ds(base + c * chunk, chunk)])

    return finalize


@jax.jit
def kernel(x, Wsel, bsel, Wexp, bexp):
    n, nin = x.shape
    nout = Wexp.shape[1]
    tb = 512

    weff, cost = pl.pallas_call(
        _routing_body,
        grid=(n // tb,),
        in_specs=[
            pl.BlockSpec((tb, nin), lambda t: (t, 0)),
            pl.BlockSpec((_E, nin), lambda t: (0, 0)),
            pl.BlockSpec((1, _E), lambda t: (0, 0)),
        ],
        out_specs=[
            pl.BlockSpec((tb, _WPAD), lambda t: (t, 0)),
            pl.BlockSpec((tb, 1), lambda t: (t, 0)),
        ],
        out_shape=[
            jax.ShapeDtypeStruct((n, _WPAD), jnp.float32),
            jax.ShapeDtypeStruct((n, 1), jnp.float32),
        ],
        compiler_params=pltpu.CompilerParams(
            dimension_semantics=("parallel",)),
    )(x, Wsel, bsel.reshape(1, _E))

    wexp_b = Wexp.astype(jnp.bfloat16)
    xb = x.astype(jnp.bfloat16)

    idx, g, cnt = _make_compact(n)(cost.reshape(n))

    def sparse_path(_):
        xc, weffc = _make_gather(n, nin)(x, weff, idx)
        xcb = xc.astype(jnp.bfloat16)
        ctb = _C
        yc = pl.pallas_call(
            _combine_body,
            grid=(_C // ctb, _E),
            in_specs=[
                pl.BlockSpec((ctb, _WPAD), lambda t, e: (t, 0)),
                pl.BlockSpec((ctb, nin), lambda t, e: (t, 0)),
                pl.BlockSpec((1, nout, nin), lambda t, e: (e, 0, 0)),
                pl.BlockSpec((1, 1, nout), lambda t, e: (e, 0, 0)),
            ],
            out_specs=pl.BlockSpec((ctb, nout), lambda t, e: (t, 0)),
            out_shape=jax.ShapeDtypeStruct((_C, nout), jnp.float32),
            compiler_params=pltpu.CompilerParams(
                dimension_semantics=("parallel", "arbitrary")),
        )(weffc, xcb, wexp_b, bexp.reshape(_E, 1, nout))
        yc_full = jnp.concatenate(
            [yc, jnp.zeros((256, nout), jnp.float32)], axis=0)
        return _make_finalize(n, nout)(yc_full, g)

    def dense_path(_):
        return _dense_combine(weff, xb, wexp_b, bexp, n, nin, nout)

    out = lax.cond(cnt[0] <= _C, sparse_path, dense_path, 0)
    return (out, cost.reshape(n))


# speedup vs baseline: 1.0017x; 1.0017x over previous
import functools

import jax
from jax import lax
import jax.numpy as jnp
from jax.experimental import pallas as pl
from jax.experimental.pallas import tpu as pltpu
from jax.experimental.pallas import tpu_sc as plsc

_E = 8
_EPS = 0.2
_LANES = 16
_WPAD = 128
_NW = 32
_C = 3584

_SORT_NET = [
    (0, 1), (2, 3), (4, 5), (6, 7),
    (0, 2), (1, 3), (4, 6), (5, 7),
    (1, 2), (5, 6), (0, 4), (3, 7),
    (1, 5), (2, 6),
    (1, 4), (3, 6),
    (2, 4), (3, 5),
    (3, 4),
]


def _routing_body(x_ref, wsel_ref, bsel_ref, weff_ref, cost_ref):
    x = x_ref[...]
    logits = jax.lax.dot_general(
        x, wsel_ref[...], (((1,), (1,)), ((), ())),
        preferred_element_type=jnp.float32) + bsel_ref[...]
    m = jnp.max(logits, axis=1, keepdims=True)
    ex = jnp.exp(logits - m)
    w = ex / jnp.sum(ex, axis=1, keepdims=True)

    ws = [w[:, j:j + 1] for j in range(_E)]
    ids = [jnp.full(ws[0].shape, j, dtype=jnp.int32) for j in range(_E)]
    for a, b in _SORT_NET:
        wa, wb = ws[a], ws[b]
        ia, ib = ids[a], ids[b]
        swap = (wb > wa) | ((wb == wa) & (ib < ia))
        ws[a] = jnp.where(swap, wb, wa)
        ws[b] = jnp.where(swap, wa, wb)
        ids[a] = jnp.where(swap, ib, ia)
        ids[b] = jnp.where(swap, ia, ib)

    cum = [ws[0]]
    for k in range(1, _E):
        cum.append(cum[-1] + ws[k])

    lim = jnp.float32(1.0 - _EPS)
    sp = []
    for k in range(_E):
        nxt = cum[k + 1] if k < _E - 1 else jnp.full(cum[0].shape, 1.0, jnp.float32)
        sp.append(jax.nn.relu(jnp.minimum(nxt, lim) - cum[k]))

    cost = sp[_E - 1]
    for k in range(_E - 1):
        cost = cost + jnp.where(sp[k + 1] > 0, jnp.float32(1.0), sp[k])

    weff_cols = []
    for k in range(_E):
        col = jnp.zeros(cum[0].shape, jnp.float32)
        for j in range(_E):
            col = jnp.where(ids[k] == j, sp[j], col)
        weff_cols.append(col)
    zero = jnp.zeros(cum[0].shape, jnp.float32)
    weff_cols.extend([zero] * (_WPAD - _E))

    weff_ref[...] = jnp.concatenate(weff_cols, axis=1)
    cost_ref[...] = cost


def _combine_body(weff_ref, x_ref, wexp_ref, bexp_ref, out_ref):
    e = pl.program_id(1)
    d = jax.lax.dot_general(
        x_ref[...], wexp_ref[0], (((1,), (1,)), ((), ())),
        preferred_element_type=jnp.float32)
    weff = weff_ref[...]
    lane = jax.lax.broadcasted_iota(jnp.int32, weff.shape, 1)
    wcol = jnp.sum(jnp.where(lane == e, weff, 0.0), axis=1, keepdims=True)
    contrib = wcol * (d + bexp_ref[0])

    @pl.when(e == 0)
    def _init():
        out_ref[...] = contrib

    @pl.when(e != 0)
    def _acc():
        out_ref[...] += contrib


def _dense_combine(weff, xb, wexp_b, bexp, n, nin, nout):
    ctb = 2048
    return pl.pallas_call(
        _combine_body,
        grid=(n // ctb, _E),
        in_specs=[
            pl.BlockSpec((ctb, _WPAD), lambda t, e: (t, 0)),
            pl.BlockSpec((ctb, nin), lambda t, e: (t, 0)),
            pl.BlockSpec((1, nout, nin), lambda t, e: (e, 0, 0)),
            pl.BlockSpec((1, 1, nout), lambda t, e: (e, 0, 0)),
        ],
        out_specs=pl.BlockSpec((ctb, nout), lambda t, e: (t, 0)),
        out_shape=jax.ShapeDtypeStruct((n, nout), jnp.float32),
        compiler_params=pltpu.CompilerParams(
            dimension_semantics=("parallel", "arbitrary")),
    )(weff, xb, wexp_b, bexp.reshape(_E, 1, nout))


def _wid():
    return lax.axis_index("s") * 2 + lax.axis_index("c")


def _sc_mesh():
    return plsc.VectorSubcoreMesh(core_axis_name="c", subcore_axis_name="s")


def _make_compact(n):
    nv = n // _LANES

    @functools.partial(
        pl.kernel,
        mesh=_sc_mesh(),
        out_type=[
            jax.ShapeDtypeStruct((_C,), jnp.int32),
            jax.ShapeDtypeStruct((n,), jnp.int32),
            jax.ShapeDtypeStruct((_LANES,), jnp.int32),
        ],
        scratch_types=[
            pltpu.VMEM((n,), jnp.float32),
            pltpu.VMEM((_C,), jnp.int32),
            pltpu.VMEM((n,), jnp.int32),
            pltpu.VMEM((_LANES,), jnp.int32),
        ],
        compiler_params=pltpu.CompilerParams(needs_layout_passes=False),
    )
    def compact(cost_hbm, idx_hbm, g_hbm, cnt_hbm, cost_v, idx_v, g_v, cnt_v):
        @pl.when(_wid() == 0)
        def _():
            pltpu.sync_copy(cost_hbm, cost_v)

            def init_body(j, carry):
                idx_v[pl.ds(j * _LANES, _LANES)] = jnp.zeros(
                    (_LANES,), jnp.int32)
                return carry
            lax.fori_loop(0, _C // _LANES, init_body, jnp.int32(0))

            def scan_body(i, offv):
                cvec = cost_v[pl.ds(i * _LANES, _LANES)]
                m = cvec > 0.0
                base = lax.broadcast(i * _LANES, (_LANES,))
                ids = lax.iota(jnp.int32, _LANES) + base
                cum = plsc.cumsum(m.astype(jnp.int32))
                pos = offv + cum - jnp.ones((_LANES,), jnp.int32)
                cap = jnp.full((_LANES,), _C, jnp.int32)
                m_ok = m & (pos < cap)
                plsc.store_scatter(idx_v, [pos], ids, mask=m_ok)
                zrow = cap + (ids & jnp.full((_LANES,), 255, jnp.int32))
                g_v[pl.ds(i * _LANES, _LANES)] = jnp.where(m_ok, pos, zrow)
                return offv + plsc.all_reduce_population_count(m)
            total = lax.fori_loop(0, nv, scan_body,
                                  jnp.zeros((_LANES,), jnp.int32))

            cnt_v[...] = total
            pltpu.sync_copy(idx_v, idx_hbm)
            pltpu.sync_copy(g_v, g_hbm)
            pltpu.sync_copy(cnt_v, cnt_hbm)

    return compact


def _make_gather(n, nin):
    pw = _C // _NW

    @functools.partial(
        pl.kernel,
        mesh=_sc_mesh(),
        out_type=[
            jax.ShapeDtypeStruct((_C, nin), jnp.float32),
            jax.ShapeDtypeStruct((_C, _WPAD), jnp.float32),
        ],
        scratch_types=[
            pltpu.VMEM((pw,), jnp.int32),
            pltpu.VMEM((pw, nin), jnp.float32),
            pltpu.VMEM((pw, _WPAD), jnp.float32),
            pltpu.SemaphoreType.DMA,
            pltpu.SemaphoreType.DMA,
        ],
    )
    def gather(x_hbm, weff_hbm, idx_hbm, xc_hbm, wc_hbm,
               idx_v, rows_v, wrow_v, sem1, sem2):
        base = _wid() * pw
        pltpu.sync_copy(idx_hbm.at[pl.ds(base, pw)], idx_v)
        cp1 = pltpu.async_copy(x_hbm.at[idx_v], rows_v, sem1)
        cp2 = pltpu.async_copy(weff_hbm.at[idx_v], wrow_v, sem2)
        cp1.wait()
        cp2.wait()
        pltpu.sync_copy(rows_v, xc_hbm.at[pl.ds(base, pw)])
        pltpu.sync_copy(wrow_v, wc_hbm.at[pl.ds(base, pw)])

    return gather


def _make_finalize(n, nout):
    pw = n // _NW
    chunk = 32
    nchunk = pw // chunk

    @functools.partial(
        pl.kernel,
        mesh=_sc_mesh(),
        out_type=jax.ShapeDtypeStruct((n, nout), jnp.float32),
        scratch_types=[
            pltpu.VMEM((pw,), jnp.int32),
            pltpu.VMEM((chunk, nout), jnp.float32),
            pltpu.VMEM((chunk, nout), jnp.float32),
            pltpu.SemaphoreType.DMA,
            pltpu.SemaphoreType.DMA,
        ],
    )
    def finalize(yc_hbm, g_hbm, out_hbm, g_v, buf0, buf1, sem0, sem1):
        base = _wid() * pw
        pltpu.sync_copy(g_hbm.at[pl.ds(base, pw)], g_v)
        bufs = (buf0, buf1)
        sems = (sem0, sem1)
        cps = [None, None]
        cps[0] = pltpu.async_copy(
            yc_hbm.at[g_v.at[pl.ds(0, chunk)]], buf0, sem0)
        for c in range(nchunk):
            nxt = c + 1
            if nxt < nchunk:
                cps[nxt % 2] = pltpu.async_copy(
                    yc_hbm.at[g_v.at[pl.ds(nxt * chunk, chunk)]],
                    bufs[nxt % 2], sems[nxt % 2])
            cps[c % 2].wait()
            pltpu.sync_copy(
                bufs[c % 2], out_hbm.at[pl.ds(base + c * chunk, chunk)])

    return finalize


@jax.jit
def kernel(x, Wsel, bsel, Wexp, bexp):
    n, nin = x.shape
    nout = Wexp.shape[1]
    tb = 512

    weff, cost = pl.pallas_call(
        _routing_body,
        grid=(n // tb,),
        in_specs=[
            pl.BlockSpec((tb, nin), lambda t: (t, 0)),
            pl.BlockSpec((_E, nin), lambda t: (0, 0)),
            pl.BlockSpec((1, _E), lambda t: (0, 0)),
        ],
        out_specs=[
            pl.BlockSpec((tb, _WPAD), lambda t: (t, 0)),
            pl.BlockSpec((tb, 1), lambda t: (t, 0)),
        ],
        out_shape=[
            jax.ShapeDtypeStruct((n, _WPAD), jnp.float32),
            jax.ShapeDtypeStruct((n, 1), jnp.float32),
        ],
        compiler_params=pltpu.CompilerParams(
            dimension_semantics=("parallel",)),
    )(x, Wsel, bsel.reshape(1, _E))

    wexp_b = Wexp.astype(jnp.bfloat16)
    xb = x.astype(jnp.bfloat16)

    idx, g, cnt = _make_compact(n)(cost.reshape(n))

    def sparse_path(_):
        xc, weffc = _make_gather(n, nin)(x, weff, idx)
        xcb = xc.astype(jnp.bfloat16)
        ctb = 1792
        yc = pl.pallas_call(
            _combine_body,
            grid=(_C // ctb, _E),
            in_specs=[
                pl.BlockSpec((ctb, _WPAD), lambda t, e: (t, 0)),
                pl.BlockSpec((ctb, nin), lambda t, e: (t, 0)),
                pl.BlockSpec((1, nout, nin), lambda t, e: (e, 0, 0)),
                pl.BlockSpec((1, 1, nout), lambda t, e: (e, 0, 0)),
            ],
            out_specs=pl.BlockSpec((ctb, nout), lambda t, e: (t, 0)),
            out_shape=jax.ShapeDtypeStruct((_C, nout), jnp.float32),
            compiler_params=pltpu.CompilerParams(
                dimension_semantics=("parallel", "arbitrary")),
        )(weffc, xcb, wexp_b, bexp.reshape(_E, 1, nout))
        yc_full = jnp.concatenate(
            [yc, jnp.zeros((256, nout), jnp.float32)], axis=0)
        return _make_finalize(n, nout)(yc_full, g)

    def dense_path(_):
        return _dense_combine(weff, xb, wexp_b, bexp, n, nin, nout)

    out = lax.cond(cnt[0] <= _C, sparse_path, dense_path, 0)
    return (out, cost.reshape(n))


# speedup vs baseline: 1.0425x; 1.0408x over previous
import functools

import jax
from jax import lax
import jax.numpy as jnp
from jax.experimental import pallas as pl
from jax.experimental.pallas import tpu as pltpu
from jax.experimental.pallas import tpu_sc as plsc

_E = 8
_EPS = 0.2
_LANES = 16
_WPAD = 128
_NW = 32
_C = 3584

_SORT_NET = [
    (0, 1), (2, 3), (4, 5), (6, 7),
    (0, 2), (1, 3), (4, 6), (5, 7),
    (1, 2), (5, 6), (0, 4), (3, 7),
    (1, 5), (2, 6),
    (1, 4), (3, 6),
    (2, 4), (3, 5),
    (3, 4),
]


def _routing_body(x_ref, wsel_ref, bsel_ref, weff_ref, cost_ref):
    x = x_ref[...]
    logits = jax.lax.dot_general(
        x, wsel_ref[...], (((1,), (1,)), ((), ())),
        preferred_element_type=jnp.float32) + bsel_ref[...]
    m = jnp.max(logits, axis=1, keepdims=True)
    ex = jnp.exp(logits - m)
    w = ex / jnp.sum(ex, axis=1, keepdims=True)

    ws = [w[:, j:j + 1] for j in range(_E)]
    ids = [jnp.full(ws[0].shape, j, dtype=jnp.int32) for j in range(_E)]
    for a, b in _SORT_NET:
        wa, wb = ws[a], ws[b]
        ia, ib = ids[a], ids[b]
        swap = (wb > wa) | ((wb == wa) & (ib < ia))
        ws[a] = jnp.where(swap, wb, wa)
        ws[b] = jnp.where(swap, wa, wb)
        ids[a] = jnp.where(swap, ib, ia)
        ids[b] = jnp.where(swap, ia, ib)

    cum = [ws[0]]
    for k in range(1, _E):
        cum.append(cum[-1] + ws[k])

    lim = jnp.float32(1.0 - _EPS)
    sp = []
    for k in range(_E):
        nxt = cum[k + 1] if k < _E - 1 else jnp.full(cum[0].shape, 1.0, jnp.float32)
        sp.append(jax.nn.relu(jnp.minimum(nxt, lim) - cum[k]))

    cost = sp[_E - 1]
    for k in range(_E - 1):
        cost = cost + jnp.where(sp[k + 1] > 0, jnp.float32(1.0), sp[k])

    weff_cols = []
    for k in range(_E):
        col = jnp.zeros(cum[0].shape, jnp.float32)
        for j in range(_E):
            col = jnp.where(ids[k] == j, sp[j], col)
        weff_cols.append(col)
    zero = jnp.zeros(cum[0].shape, jnp.float32)
    weff_cols.extend([zero] * (_WPAD - _E))

    weff_ref[...] = jnp.concatenate(weff_cols, axis=1)
    cost_ref[...] = cost


def _combine_body(weff_ref, x_ref, wexp_ref, bexp_ref, out_ref):
    e = pl.program_id(1)
    d = jax.lax.dot_general(
        x_ref[...], wexp_ref[0], (((1,), (1,)), ((), ())),
        preferred_element_type=jnp.float32)
    weff = weff_ref[...]
    lane = jax.lax.broadcasted_iota(jnp.int32, weff.shape, 1)
    wcol = jnp.sum(jnp.where(lane == e, weff, 0.0), axis=1, keepdims=True)
    contrib = wcol * (d + bexp_ref[0])

    @pl.when(e == 0)
    def _init():
        out_ref[...] = contrib

    @pl.when(e != 0)
    def _acc():
        out_ref[...] += contrib


def _dense_combine(weff, xb, wexp_b, bexp, n, nin, nout):
    ctb = 2048
    return pl.pallas_call(
        _combine_body,
        grid=(n // ctb, _E),
        in_specs=[
            pl.BlockSpec((ctb, _WPAD), lambda t, e: (t, 0)),
            pl.BlockSpec((ctb, nin), lambda t, e: (t, 0)),
            pl.BlockSpec((1, nout, nin), lambda t, e: (e, 0, 0)),
            pl.BlockSpec((1, 1, nout), lambda t, e: (e, 0, 0)),
        ],
        out_specs=pl.BlockSpec((ctb, nout), lambda t, e: (t, 0)),
        out_shape=jax.ShapeDtypeStruct((n, nout), jnp.float32),
        compiler_params=pltpu.CompilerParams(
            dimension_semantics=("parallel", "arbitrary")),
    )(weff, xb, wexp_b, bexp.reshape(_E, 1, nout))


def _wid():
    return lax.axis_index("s") * 2 + lax.axis_index("c")


def _sc_mesh():
    return plsc.VectorSubcoreMesh(core_axis_name="c", subcore_axis_name="s")


def _make_compact(n):
    nv = n // _LANES

    @functools.partial(
        pl.kernel,
        mesh=_sc_mesh(),
        out_type=[
            jax.ShapeDtypeStruct((_C,), jnp.int32),
            jax.ShapeDtypeStruct((n,), jnp.int32),
            jax.ShapeDtypeStruct((_LANES,), jnp.int32),
        ],
        scratch_types=[
            pltpu.VMEM((n,), jnp.float32),
            pltpu.VMEM((_C,), jnp.int32),
            pltpu.VMEM((n,), jnp.int32),
            pltpu.VMEM((_LANES,), jnp.int32),
        ],
        compiler_params=pltpu.CompilerParams(needs_layout_passes=False),
    )
    def compact(cost_hbm, idx_hbm, g_hbm, cnt_hbm, cost_v, idx_v, g_v, cnt_v):
        @pl.when(_wid() == 0)
        def _():
            pltpu.sync_copy(cost_hbm, cost_v)

            def init_body(j, carry):
                idx_v[pl.ds(j * _LANES, _LANES)] = jnp.zeros(
                    (_LANES,), jnp.int32)
                return carry
            lax.fori_loop(0, _C // _LANES, init_body, jnp.int32(0))

            def scan_body(i, offv):
                cvec = cost_v[pl.ds(i * _LANES, _LANES)]
                m = cvec > 0.0
                base = lax.broadcast(i * _LANES, (_LANES,))
                ids = lax.iota(jnp.int32, _LANES) + base
                cum = plsc.cumsum(m.astype(jnp.int32))
                pos = offv + cum - jnp.ones((_LANES,), jnp.int32)
                cap = jnp.full((_LANES,), _C, jnp.int32)
                m_ok = m & (pos < cap)
                plsc.store_scatter(idx_v, [pos], ids, mask=m_ok)
                zrow = cap + (ids & jnp.full((_LANES,), 255, jnp.int32))
                g_v[pl.ds(i * _LANES, _LANES)] = jnp.where(m_ok, pos, zrow)
                return offv + plsc.all_reduce_population_count(m)
            total = lax.fori_loop(0, nv, scan_body,
                                  jnp.zeros((_LANES,), jnp.int32))

            cnt_v[...] = total
            pltpu.sync_copy(idx_v, idx_hbm)
            pltpu.sync_copy(g_v, g_hbm)
            pltpu.sync_copy(cnt_v, cnt_hbm)

    return compact


def _make_gather(n, nin):
    pw = _C // _NW

    @functools.partial(
        pl.kernel,
        mesh=_sc_mesh(),
        out_type=[
            jax.ShapeDtypeStruct((_C, nin), jnp.float32),
            jax.ShapeDtypeStruct((_C, _WPAD), jnp.float32),
        ],
        scratch_types=[
            pltpu.VMEM((pw,), jnp.int32),
            pltpu.VMEM((pw, nin), jnp.float32),
            pltpu.VMEM((pw, _WPAD), jnp.float32),
            pltpu.SemaphoreType.DMA,
            pltpu.SemaphoreType.DMA,
        ],
    )
    def gather(x_hbm, weff_hbm, idx_hbm, xc_hbm, wc_hbm,
               idx_v, rows_v, wrow_v, sem1, sem2):
        base = _wid() * pw
        pltpu.sync_copy(idx_hbm.at[pl.ds(base, pw)], idx_v)
        cp1 = pltpu.async_copy(x_hbm.at[idx_v], rows_v, sem1)
        cp2 = pltpu.async_copy(weff_hbm.at[idx_v], wrow_v, sem2)
        cp1.wait()
        cp2.wait()
        pltpu.sync_copy(rows_v, xc_hbm.at[pl.ds(base, pw)])
        pltpu.sync_copy(wrow_v, wc_hbm.at[pl.ds(base, pw)])

    return gather


def _make_finalize(n, nout):
    pw = n // _NW
    chunk = 32
    nchunk = pw // chunk

    @functools.partial(
        pl.kernel,
        mesh=_sc_mesh(),
        out_type=jax.ShapeDtypeStruct((n, nout), jnp.float32),
        scratch_types=[
            pltpu.VMEM((pw,), jnp.int32),
            pltpu.VMEM((chunk, nout), jnp.float32),
            pltpu.VMEM((chunk, nout), jnp.float32),
            pltpu.SemaphoreType.DMA,
            pltpu.SemaphoreType.DMA,
        ],
    )
    def finalize(yc_hbm, g_hbm, out_hbm, g_v, buf0, buf1, sem0, sem1):
        base = _wid() * pw
        pltpu.sync_copy(g_hbm.at[pl.ds(base, pw)], g_v)
        bufs = (buf0, buf1)
        sems = (sem0, sem1)
        cps = [None, None]
        cps[0] = pltpu.async_copy(
            yc_hbm.at[g_v.at[pl.ds(0, chunk)]], buf0, sem0)
        for c in range(nchunk):
            nxt = c + 1
            if nxt < nchunk:
                cps[nxt % 2] = pltpu.async_copy(
                    yc_hbm.at[g_v.at[pl.ds(nxt * chunk, chunk)]],
                    bufs[nxt % 2], sems[nxt % 2])
            cps[c % 2].wait()
            pltpu.sync_copy(
                bufs[c % 2], out_hbm.at[pl.ds(base + c * chunk, chunk)])

    return finalize


@jax.jit
def kernel(x, Wsel, bsel, Wexp, bexp):
    n, nin = x.shape
    nout = Wexp.shape[1]
    tb = 512

    weff, cost = pl.pallas_call(
        _routing_body,
        grid=(n // tb,),
        in_specs=[
            pl.BlockSpec((tb, nin), lambda t: (t, 0)),
            pl.BlockSpec((_E, nin), lambda t: (0, 0)),
            pl.BlockSpec((1, _E), lambda t: (0, 0)),
        ],
        out_specs=[
            pl.BlockSpec((tb, _WPAD), lambda t: (t, 0)),
            pl.BlockSpec((tb, 1), lambda t: (t, 0)),
        ],
        out_shape=[
            jax.ShapeDtypeStruct((n, _WPAD), jnp.float32),
            jax.ShapeDtypeStruct((n, 1), jnp.float32),
        ],
        compiler_params=pltpu.CompilerParams(
            dimension_semantics=("parallel",)),
    )(x, Wsel, bsel.reshape(1, _E))

    wexp_b = Wexp.astype(jnp.bfloat16)

    idx, g, cnt = _make_compact(n)(cost.reshape(n))

    def sparse_path(_):
        xc, weffc = _make_gather(n, nin)(x, weff, idx)
        xcb = xc.astype(jnp.bfloat16)
        ctb = 1792
        yc = pl.pallas_call(
            _combine_body,
            grid=(_C // ctb, _E),
            in_specs=[
                pl.BlockSpec((ctb, _WPAD), lambda t, e: (t, 0)),
                pl.BlockSpec((ctb, nin), lambda t, e: (t, 0)),
                pl.BlockSpec((1, nout, nin), lambda t, e: (e, 0, 0)),
                pl.BlockSpec((1, 1, nout), lambda t, e: (e, 0, 0)),
            ],
            out_specs=pl.BlockSpec((ctb, nout), lambda t, e: (t, 0)),
            out_shape=jax.ShapeDtypeStruct((_C, nout), jnp.float32),
            compiler_params=pltpu.CompilerParams(
                dimension_semantics=("parallel", "arbitrary")),
        )(weffc, xcb, wexp_b, bexp.reshape(_E, 1, nout))
        yc_full = jnp.concatenate(
            [yc, jnp.zeros((256, nout), jnp.float32)], axis=0)
        return _make_finalize(n, nout)(yc_full, g)

    def dense_path(_):
        xb = x.astype(jnp.bfloat16)
        return _dense_combine(weff, xb, wexp_b, bexp, n, nin, nout)

    out = lax.cond(cnt[0] <= _C, sparse_path, dense_path, 0)
    return (out, cost.reshape(n))
